# baseline (speedup 1.0000x reference)
import functools

import jax
import jax.numpy as jnp
from jax import lax
from jax.experimental import pallas as pl
from jax.experimental.pallas import tpu as pltpu
from jax.experimental.pallas import tpu_sc as plsc

V = 10000
E = 160000
B = 64
K = 5
HID = 256
OUT = 40
NUM_LAYERS = 3

NC = 2
NS = 16
NW = NC * NS
CHUNK = 128
NCHUNKS = E // CHUNK
NJ = NCHUNKS // NW
NREM = NCHUNKS - NJ * NW
ROWS_PER_TILE = 624
TAIL_BASE = NS * ROWS_PER_TILE
TAIL_ROWS = V - TAIL_BASE


def _spmm_body(mat_hbm, src_hbm, dst_hbm, vals_hbm, zeros_hbm, out_hbm,
               src_t, dst_t, vals_t, rows0, rows1, rows2, rows3,
               gsem0, gsem1, gsem2, gsem3, ssem0, ssem1, ssem2, ssem3,
               psem, zsem, acc_sh):
    cid = lax.axis_index("c")
    sid = lax.axis_index("s")
    wid = cid * NS + sid

    row0 = wid * NJ
    _pre = ((src_hbm, src_t), (dst_hbm, dst_t), (vals_hbm, vals_t))
    for hb, tt in _pre:
        pltpu.async_copy(hb.at[pl.ds(row0, NJ)], tt.at[pl.ds(0, NJ)], psem)
    pltpu.async_copy(
        zeros_hbm.at[pl.ds(0, ROWS_PER_TILE)],
        acc_sh.at[pl.ds(sid * ROWS_PER_TILE, ROWS_PER_TILE)], zsem)
    @pl.when(wid < NREM)
    def _():
        tr = NJ * NW + wid
        for hb, tt in _pre:
            pltpu.async_copy(hb.at[pl.ds(tr, 1)], tt.at[pl.ds(NJ, 1)], psem)
    @pl.when(sid == NS - 1)
    def _():
        pltpu.async_copy(zeros_hbm.at[pl.ds(0, TAIL_ROWS)],
                         acc_sh.at[pl.ds(TAIL_BASE, TAIL_ROWS)], zsem)
    nr = NJ + jnp.where(wid < NREM, 1, 0)
    for hb, tt in _pre:
        pltpu.make_async_copy(hb.at[pl.ds(row0, NJ)],
                              tt.at[pl.ds(0, NJ)], psem).wait()
    @pl.when(wid < NREM)
    def _():
        tr = NJ * NW + wid
        for hb, tt in _pre:
            pltpu.make_async_copy(hb.at[pl.ds(tr, 1)],
                                  tt.at[pl.ds(NJ, 1)], psem).wait()

    bufs = ((rows0, gsem0, ssem0), (rows1, gsem1, ssem1),
            (rows2, gsem2, ssem2), (rows3, gsem3, ssem3))

    def g_issue(j, b):
        pltpu.async_copy(mat_hbm.at[src_t.at[j]], bufs[b][0], bufs[b][1])

    def g_wait(j, b):
        pltpu.make_async_copy(
            mat_hbm.at[src_t.at[j]], bufs[b][0], bufs[b][1]).wait()

    def s_issue(j, b):
        pltpu.async_copy(bufs[b][0], acc_sh.at[dst_t.at[j]], bufs[b][2],
                         add=True)

    def s_wait(j, b):
        pltpu.make_async_copy(
            bufs[b][0], acc_sh.at[dst_t.at[j]], bufs[b][2]).wait()

    def scale(j, b):
        rows = bufs[b][0]
        @plsc.parallel_loop(0, CHUNK // 16, 1, unroll=2)
        def _(g):
            for l in range(16):
                bval = plsc.load_gather(
                    vals_t, [jnp.broadcast_to(j, (16,)),
                             jnp.broadcast_to(g * 16 + l, (16,))])
                e = g * 16 + l
                for s in range(B // 16):
                    rows[e, pl.ds(s * 16, 16)] = (
                        rows[e, pl.ds(s * 16, 16)] * bval)

    g_issue(0, 0)
    g_issue(1, 1)

    pltpu.make_async_copy(
        zeros_hbm.at[pl.ds(0, ROWS_PER_TILE)],
        acc_sh.at[pl.ds(sid * ROWS_PER_TILE, ROWS_PER_TILE)], zsem).wait()
    @pl.when(sid == NS - 1)
    def _():
        pltpu.make_async_copy(zeros_hbm.at[pl.ds(0, TAIL_ROWS)],
                              acc_sh.at[pl.ds(TAIL_BASE, TAIL_ROWS)],
                              zsem).wait()
    plsc.subcore_barrier()

    def quad_body(i, carry):
        for b in range(4):
            j = i * 4 + b
            @pl.when(j < nr)
            def _():
                g_wait(j, b)
                scale(j, b)
                s_issue(j, b)
                jf = j + 2
                bf = (b + 2) % 4
                @pl.when(jf < nr)
                def _():
                    @pl.when(jf >= 4)
                    def _():
                        s_wait(jf - 4, bf)
                    g_issue(jf, bf)
        return carry
    lax.fori_loop(0, (NJ + 4) // 4, quad_body, 0)

    for b in range(4):
        s_wait(0, b)

    plsc.subcore_barrier()
    pltpu.sync_copy(acc_sh.at[pl.ds(sid * ROWS_PER_TILE, ROWS_PER_TILE)],
                    out_hbm.at[cid, pl.ds(sid * ROWS_PER_TILE, ROWS_PER_TILE)])
    @pl.when(sid == NS - 1)
    def _():
        pltpu.sync_copy(acc_sh.at[pl.ds(TAIL_BASE, TAIL_ROWS)],
                        out_hbm.at[cid, pl.ds(TAIL_BASE, TAIL_ROWS)])


def _spmm_partials(mat, src, dst, vals, zeros):
    f = functools.partial(
        pl.kernel,
        out_type=jax.ShapeDtypeStruct((NC, V, B), jnp.float32),
        mesh=plsc.VectorSubcoreMesh(core_axis_name="c", subcore_axis_name="s",
                                    num_cores=NC, num_subcores=NS),
        compiler_params=pltpu.CompilerParams(needs_layout_passes=False,
                                             use_tc_tiling_on_sc=False),
        scratch_types=[
            pltpu.VMEM((NJ + 1, CHUNK), jnp.int32),
            pltpu.VMEM((NJ + 1, CHUNK), jnp.int32),
            pltpu.VMEM((NJ + 1, CHUNK), jnp.float32),
            pltpu.VMEM((CHUNK, B), jnp.float32),
            pltpu.VMEM((CHUNK, B), jnp.float32),
            pltpu.VMEM((CHUNK, B), jnp.float32),
            pltpu.VMEM((CHUNK, B), jnp.float32),
            pltpu.SemaphoreType.DMA,
            pltpu.SemaphoreType.DMA,
            pltpu.SemaphoreType.DMA,
            pltpu.SemaphoreType.DMA,
            pltpu.SemaphoreType.DMA,
            pltpu.SemaphoreType.DMA,
            pltpu.SemaphoreType.DMA,
            pltpu.SemaphoreType.DMA,
            pltpu.SemaphoreType.DMA,
            pltpu.SemaphoreType.DMA,
            pltpu.VMEM_SHARED((V, B), jnp.float32),
        ],
    )(_spmm_body)
    return f(mat, src, dst, vals, zeros)


VR = V * B // 128


def _combine1_body(p_ref, x_ref, o_ref):
    o_ref[...] = p_ref[0] + p_ref[1] - x_ref[...]


def _combine2_body(p_ref, x1_ref, x0_ref, o_ref):
    o_ref[...] = 2.0 * (p_ref[0] + p_ref[1]) - 2.0 * x1_ref[...] - x0_ref[...]


def _combine1(p, x0):
    return pl.pallas_call(
        _combine1_body,
        out_shape=jax.ShapeDtypeStruct((VR, 128), jnp.float32),
    )(p.reshape(2, VR, 128), x0)


def _combine2(p, x1, x0):
    return pl.pallas_call(
        _combine2_body,
        out_shape=jax.ShapeDtypeStruct((VR, 128), jnp.float32),
    )(p.reshape(2, VR, 128), x1, x0)


def _transpose_body(x_ref, o_ref):
    xt = jnp.transpose(x_ref[...], (1, 0))
    a = xt.reshape(VR, 2, B)
    o_ref[...] = jnp.concatenate([a[:, 0, :], a[:, 1, :]], axis=1)


def _transpose(x):
    return pl.pallas_call(
        _transpose_body,
        out_shape=jax.ShapeDtypeStruct((VR, 128), jnp.float32),
    )(x)


def _dotT(a_pb, w_pn):
    return lax.dot_general(a_pb, w_pn, (((0,), (0,)), ((), ())),
                           preferred_element_type=jnp.float32)


def _epilogue_body(x0, x1, x2, x3, p4, Wc, bc, Wf, bf, gam, bet,
                   Wl1, bl1, Wl2, bl2, o_ref, y_ref):
    xv = [x[...] for x in (x0, x1, x2, x3)]
    xv.append(2.0 * (p4[0] + p4[1]) - 2.0 * xv[3] - xv[2])
    hps = []
    for f in range(K):
        h = jnp.zeros((VR, 128), jnp.float32) + bc[f]
        for k in range(K):
            h = h + Wc[k, f] * xv[k]
        h = jnp.maximum(h, 0.0)
        red = jnp.max(h.reshape(1250, 4, 128), axis=1)
        hp = jnp.maximum(red[:, :B], red[:, B:])
        hps.append(hp.reshape(10, 125, B))
    hpall = jnp.concatenate(hps, axis=1)
    for c in range(10):
        y_ref[c] = _dotT(hpall[c], Wf[...]) + bf[...]
    z = y_ref[...].reshape(10 * B, HID)
    xj = None
    for i in range(NUM_LAYERS):
        mean = jnp.mean(z, axis=0)
        var = jnp.mean((z - mean[None, :]) ** 2, axis=0)
        z = (z - mean[None, :]) * lax.rsqrt(var[None, :] + 1e-5)
        z = z * gam[i][None, :] + bet[i][None, :]
        z = jnp.maximum(z, 0.0)
        xj = z if xj is None else jnp.maximum(xj, z)
    o = jnp.dot(xj, Wl1[...], preferred_element_type=jnp.float32) + bl1[...][None, :]
    o = jnp.maximum(o, 0.0)
    o = jnp.dot(o, Wl2[...], preferred_element_type=jnp.float32) + bl2[...][None, :]
    m = jnp.max(o, axis=1, keepdims=True)
    lse = jnp.log(jnp.sum(jnp.exp(o - m), axis=1, keepdims=True)) + m
    o_ref[...] = o - lse


def _epilogue(xstack, Wc, bc, Wf, bf, gam, bet, Wl1, bl1, Wl2, bl2):
    smem = pl.BlockSpec(memory_space=pltpu.SMEM)
    vmem = pl.BlockSpec(memory_space=pltpu.VMEM)
    return pl.pallas_call(
        _epilogue_body,
        out_shape=jax.ShapeDtypeStruct((10 * B, OUT), jnp.float32),
        in_specs=[vmem] * 5 + [smem, smem] + [vmem] * 8,
        out_specs=vmem,
        scratch_shapes=[pltpu.VMEM((10, B, HID), jnp.float32)],
    )(*xstack, Wc, bc, Wf, bf, gam, bet, Wl1, bl1, Wl2, bl2)


def kernel(x, L_indices, L_values, W_cl1, b_cl1, W_fc1, b_fc1,
           bn_gamma, bn_beta, W_lin1, b_lin1, W_lin2, b_lin2):
    src = L_indices[1].reshape(NCHUNKS, CHUNK)
    dst = L_indices[0].reshape(NCHUNKS, CHUNK)
    L_values = L_values.reshape(NCHUNKS, CHUNK)
    zeros = jnp.zeros((ROWS_PER_TILE, B), jnp.float32)

    x0 = jnp.transpose(x, (1, 0))
    x0v = x0.reshape(VR, 128)
    p = _spmm_partials(x0, src, dst, L_values, zeros)
    x1v = _combine1(p, x0v)
    p = _spmm_partials(x1v.reshape(V, B), src, dst, L_values, zeros)
    x2v = _combine2(p, x1v, x0v)
    p = _spmm_partials(x2v.reshape(V, B), src, dst, L_values, zeros)
    x3v = _combine2(p, x2v, x1v)
    p4 = _spmm_partials(x3v.reshape(V, B), src, dst, L_values, zeros)

    Wf = jnp.transpose(W_fc1.reshape(125, K, HID), (1, 0, 2)).reshape(
        K * 125, HID)
    o = _epilogue((x0v, x1v, x2v, x3v, p4.reshape(2, VR, 128)),
                  W_cl1, b_cl1, Wf, b_fc1,
                  bn_gamma, bn_beta, W_lin1, b_lin1, W_lin2, b_lin2)
    return o.reshape(10, B, OUT).transpose(1, 0, 2).reshape(10 * B, OUT)


# speedup vs baseline: 1.0316x; 1.0316x over previous
import functools

import jax
import jax.numpy as jnp
from jax import lax
from jax.experimental import pallas as pl
from jax.experimental.pallas import tpu as pltpu
from jax.experimental.pallas import tpu_sc as plsc

V = 10000
E = 160000
B = 64
K = 5
HID = 256
OUT = 40
NUM_LAYERS = 3

NC = 2
NS = 16
NW = NC * NS
CHUNK = 128
NCHUNKS = E // CHUNK
NJ = NCHUNKS // NW
NREM = NCHUNKS - NJ * NW
ROWS_PER_TILE = 624
TAIL_BASE = NS * ROWS_PER_TILE
TAIL_ROWS = V - TAIL_BASE


def _spmm_body(mat_hbm, src_hbm, dst_hbm, vals_hbm, zeros_hbm, out_hbm,
               src_t, dst_t, vals_t, rows0, rows1, rows2, rows3,
               gsem0, gsem1, gsem2, gsem3, ssem0, ssem1, ssem2, ssem3,
               psem, zsem, acc_sh):
    cid = lax.axis_index("c")
    sid = lax.axis_index("s")
    wid = cid * NS + sid

    row0 = wid * NJ
    _pre = ((src_hbm, src_t), (dst_hbm, dst_t), (vals_hbm, vals_t))
    for hb, tt in _pre:
        pltpu.async_copy(hb.at[pl.ds(row0, NJ)], tt.at[pl.ds(0, NJ)], psem)
    pltpu.async_copy(
        zeros_hbm.at[pl.ds(sid * ROWS_PER_TILE, ROWS_PER_TILE)],
        acc_sh.at[pl.ds(sid * ROWS_PER_TILE, ROWS_PER_TILE)], zsem)
    @pl.when(wid < NREM)
    def _():
        tr = NJ * NW + wid
        for hb, tt in _pre:
            pltpu.async_copy(hb.at[pl.ds(tr, 1)], tt.at[pl.ds(NJ, 1)], psem)
    @pl.when(sid == NS - 1)
    def _():
        pltpu.async_copy(zeros_hbm.at[pl.ds(TAIL_BASE, TAIL_ROWS)],
                         acc_sh.at[pl.ds(TAIL_BASE, TAIL_ROWS)], zsem)
    nr = NJ + jnp.where(wid < NREM, 1, 0)
    for hb, tt in _pre:
        pltpu.make_async_copy(hb.at[pl.ds(row0, NJ)],
                              tt.at[pl.ds(0, NJ)], psem).wait()
    @pl.when(wid < NREM)
    def _():
        tr = NJ * NW + wid
        for hb, tt in _pre:
            pltpu.make_async_copy(hb.at[pl.ds(tr, 1)],
                                  tt.at[pl.ds(NJ, 1)], psem).wait()

    bufs = ((rows0, gsem0, ssem0), (rows1, gsem1, ssem1),
            (rows2, gsem2, ssem2), (rows3, gsem3, ssem3))

    def g_issue(j, b):
        pltpu.async_copy(mat_hbm.at[src_t.at[j]], bufs[b][0], bufs[b][1])

    def g_wait(j, b):
        pltpu.make_async_copy(
            mat_hbm.at[src_t.at[j]], bufs[b][0], bufs[b][1]).wait()

    def s_issue(j, b):
        pltpu.async_copy(bufs[b][0], acc_sh.at[dst_t.at[j]], bufs[b][2],
                         add=True)

    def s_wait(j, b):
        pltpu.make_async_copy(
            bufs[b][0], acc_sh.at[dst_t.at[j]], bufs[b][2]).wait()

    def scale(j, b):
        rows = bufs[b][0]
        @plsc.parallel_loop(0, CHUNK // 16, 1, unroll=2)
        def _(g):
            for l in range(16):
                bval = plsc.load_gather(
                    vals_t, [jnp.broadcast_to(j, (16,)),
                             jnp.broadcast_to(g * 16 + l, (16,))])
                e = g * 16 + l
                for s in range(B // 16):
                    rows[e, pl.ds(s * 16, 16)] = (
                        rows[e, pl.ds(s * 16, 16)] * bval)

    g_issue(0, 0)
    g_issue(1, 1)

    pltpu.make_async_copy(
        zeros_hbm.at[pl.ds(sid * ROWS_PER_TILE, ROWS_PER_TILE)],
        acc_sh.at[pl.ds(sid * ROWS_PER_TILE, ROWS_PER_TILE)], zsem).wait()
    @pl.when(sid == NS - 1)
    def _():
        pltpu.make_async_copy(zeros_hbm.at[pl.ds(TAIL_BASE, TAIL_ROWS)],
                              acc_sh.at[pl.ds(TAIL_BASE, TAIL_ROWS)],
                              zsem).wait()
    plsc.subcore_barrier()

    def quad_body(i, carry):
        for b in range(4):
            j = i * 4 + b
            @pl.when(j < nr)
            def _():
                g_wait(j, b)
                scale(j, b)
                s_issue(j, b)
                jf = j + 2
                bf = (b + 2) % 4
                @pl.when(jf < nr)
                def _():
                    @pl.when(jf >= 4)
                    def _():
                        s_wait(jf - 4, bf)
                    g_issue(jf, bf)
        return carry
    lax.fori_loop(0, (NJ + 4) // 4, quad_body, 0)

    for b in range(4):
        s_wait(0, b)

    plsc.subcore_barrier()
    pltpu.sync_copy(acc_sh.at[pl.ds(sid * ROWS_PER_TILE, ROWS_PER_TILE)],
                    out_hbm.at[cid, pl.ds(sid * ROWS_PER_TILE, ROWS_PER_TILE)])
    @pl.when(sid == NS - 1)
    def _():
        pltpu.sync_copy(acc_sh.at[pl.ds(TAIL_BASE, TAIL_ROWS)],
                        out_hbm.at[cid, pl.ds(TAIL_BASE, TAIL_ROWS)])


def _spmm_partials(mat, src, dst, vals, zeros):
    f = functools.partial(
        pl.kernel,
        out_type=jax.ShapeDtypeStruct((NC, V, B), jnp.float32),
        mesh=plsc.VectorSubcoreMesh(core_axis_name="c", subcore_axis_name="s",
                                    num_cores=NC, num_subcores=NS),
        compiler_params=pltpu.CompilerParams(needs_layout_passes=False,
                                             use_tc_tiling_on_sc=False),
        scratch_types=[
            pltpu.VMEM((NJ + 1, CHUNK), jnp.int32),
            pltpu.VMEM((NJ + 1, CHUNK), jnp.int32),
            pltpu.VMEM((NJ + 1, CHUNK), jnp.float32),
            pltpu.VMEM((CHUNK, B), jnp.float32),
            pltpu.VMEM((CHUNK, B), jnp.float32),
            pltpu.VMEM((CHUNK, B), jnp.float32),
            pltpu.VMEM((CHUNK, B), jnp.float32),
            pltpu.SemaphoreType.DMA,
            pltpu.SemaphoreType.DMA,
            pltpu.SemaphoreType.DMA,
            pltpu.SemaphoreType.DMA,
            pltpu.SemaphoreType.DMA,
            pltpu.SemaphoreType.DMA,
            pltpu.SemaphoreType.DMA,
            pltpu.SemaphoreType.DMA,
            pltpu.SemaphoreType.DMA,
            pltpu.SemaphoreType.DMA,
            pltpu.VMEM_SHARED((V, B), jnp.float32),
        ],
    )(_spmm_body)
    return f(mat, src, dst, vals, zeros)


VR = V * B // 128


def _combine1_body(p_ref, x_ref, o_ref):
    o_ref[...] = p_ref[0] + p_ref[1] - x_ref[...]


def _combine2_body(p_ref, x1_ref, x0_ref, o_ref):
    o_ref[...] = 2.0 * (p_ref[0] + p_ref[1]) - 2.0 * x1_ref[...] - x0_ref[...]


def _combine1(p, x0):
    return pl.pallas_call(
        _combine1_body,
        out_shape=jax.ShapeDtypeStruct((VR, 128), jnp.float32),
    )(p.reshape(2, VR, 128), x0)


def _combine2(p, x1, x0):
    return pl.pallas_call(
        _combine2_body,
        out_shape=jax.ShapeDtypeStruct((VR, 128), jnp.float32),
    )(p.reshape(2, VR, 128), x1, x0)


def _transpose_body(x_ref, o_ref):
    xt = jnp.transpose(x_ref[...], (1, 0))
    a = xt.reshape(VR, 2, B)
    o_ref[...] = jnp.concatenate([a[:, 0, :], a[:, 1, :]], axis=1)


def _transpose(x):
    return pl.pallas_call(
        _transpose_body,
        out_shape=jax.ShapeDtypeStruct((VR, 128), jnp.float32),
    )(x)


def _dotT(a_pb, w_pn):
    return lax.dot_general(a_pb, w_pn, (((0,), (0,)), ((), ())),
                           preferred_element_type=jnp.float32)


def _epilogue_body(x0, x1, x2, x3, p4, Wc, bc, Wf, bf, gam, bet,
                   Wl1, bl1, Wl2, bl2, o_ref, y_ref):
    xv = [x[...] for x in (x0, x1, x2, x3)]
    xv.append(2.0 * (p4[0] + p4[1]) - 2.0 * xv[3] - xv[2])
    hps = []
    for f in range(K):
        h = jnp.zeros((VR, 128), jnp.float32) + bc[f]
        for k in range(K):
            h = h + Wc[k, f] * xv[k]
        h = jnp.maximum(h, 0.0)
        red = jnp.max(h.reshape(1250, 4, 128), axis=1)
        hp = jnp.maximum(red[:, :B], red[:, B:])
        hps.append(hp.reshape(10, 125, B))
    hpall = jnp.concatenate(hps, axis=1)
    for c in range(10):
        y_ref[c] = _dotT(hpall[c], Wf[...]) + bf[...]
    z = y_ref[...].reshape(10 * B, HID)
    xj = None
    for i in range(NUM_LAYERS):
        mean = jnp.mean(z, axis=0)
        var = jnp.mean((z - mean[None, :]) ** 2, axis=0)
        z = (z - mean[None, :]) * lax.rsqrt(var[None, :] + 1e-5)
        z = z * gam[i][None, :] + bet[i][None, :]
        z = jnp.maximum(z, 0.0)
        xj = z if xj is None else jnp.maximum(xj, z)
    o = jnp.dot(xj, Wl1[...], preferred_element_type=jnp.float32) + bl1[...][None, :]
    o = jnp.maximum(o, 0.0)
    o = jnp.dot(o, Wl2[...], preferred_element_type=jnp.float32) + bl2[...][None, :]
    m = jnp.max(o, axis=1, keepdims=True)
    lse = jnp.log(jnp.sum(jnp.exp(o - m), axis=1, keepdims=True)) + m
    o_ref[...] = o - lse


def _epilogue(xstack, Wc, bc, Wf, bf, gam, bet, Wl1, bl1, Wl2, bl2):
    smem = pl.BlockSpec(memory_space=pltpu.SMEM)
    vmem = pl.BlockSpec(memory_space=pltpu.VMEM)
    return pl.pallas_call(
        _epilogue_body,
        out_shape=jax.ShapeDtypeStruct((10 * B, OUT), jnp.float32),
        in_specs=[vmem] * 5 + [smem, smem] + [vmem] * 8,
        out_specs=vmem,
        scratch_shapes=[pltpu.VMEM((10, B, HID), jnp.float32)],
    )(*xstack, Wc, bc, Wf, bf, gam, bet, Wl1, bl1, Wl2, bl2)


def kernel(x, L_indices, L_values, W_cl1, b_cl1, W_fc1, b_fc1,
           bn_gamma, bn_beta, W_lin1, b_lin1, W_lin2, b_lin2):
    src = L_indices[1].reshape(NCHUNKS, CHUNK)
    dst = L_indices[0].reshape(NCHUNKS, CHUNK)
    L_values = L_values.reshape(NCHUNKS, CHUNK)
    zeros = jnp.zeros((V, B), jnp.float32)

    x0 = jnp.transpose(x, (1, 0))
    x0v = x0.reshape(VR, 128)
    p = _spmm_partials(x0, src, dst, L_values, zeros)
    x1v = _combine1(p, x0v)
    p = _spmm_partials(x1v.reshape(V, B), src, dst, L_values, zeros)
    x2v = _combine2(p, x1v, x0v)
    p = _spmm_partials(x2v.reshape(V, B), src, dst, L_values, zeros)
    x3v = _combine2(p, x2v, x1v)
    p4 = _spmm_partials(x3v.reshape(V, B), src, dst, L_values, zeros)

    Wf = jnp.transpose(W_fc1.reshape(125, K, HID), (1, 0, 2)).reshape(
        K * 125, HID)
    o = _epilogue((x0v, x1v, x2v, x3v, p4.reshape(2, VR, 128)),
                  W_cl1, b_cl1, Wf, b_fc1,
                  bn_gamma, bn_beta, W_lin1, b_lin1, W_lin2, b_lin2)
    return o.reshape(10, B, OUT).transpose(1, 0, 2).reshape(10 * B, OUT)


# speedup vs baseline: 1.0321x; 1.0005x over previous
import functools

import jax
import jax.numpy as jnp
from jax import lax
from jax.experimental import pallas as pl
from jax.experimental.pallas import tpu as pltpu
from jax.experimental.pallas import tpu_sc as plsc

V = 10000
E = 160000
B = 64
K = 5
HID = 256
OUT = 40
NUM_LAYERS = 3

NC = 2
NS = 16
NW = NC * NS
CHUNK = 128
NCHUNKS = E // CHUNK
NJ = NCHUNKS // NW
NREM = NCHUNKS - NJ * NW
ROWS_PER_TILE = 624
TAIL_BASE = NS * ROWS_PER_TILE
TAIL_ROWS = V - TAIL_BASE


def _spmm_body(mat_hbm, src_hbm, dst_hbm, vals_hbm, zeros_hbm, out_hbm,
               src_t, dst_t, vals_t, rows0, rows1, rows2, rows3,
               gsem0, gsem1, gsem2, gsem3, ssem0, ssem1, ssem2, ssem3,
               psem, zsem, acc_sh):
    cid = lax.axis_index("c")
    sid = lax.axis_index("s")
    wid = cid * NS + sid

    row0 = wid * NJ
    _pre = ((src_hbm, src_t), (dst_hbm, dst_t), (vals_hbm, vals_t))
    for hb, tt in _pre:
        pltpu.async_copy(hb.at[pl.ds(row0, NJ)], tt.at[pl.ds(0, NJ)], psem)
    pltpu.async_copy(
        zeros_hbm.at[pl.ds(sid * ROWS_PER_TILE, ROWS_PER_TILE)],
        acc_sh.at[pl.ds(sid * ROWS_PER_TILE, ROWS_PER_TILE)], zsem)
    @pl.when(wid < NREM)
    def _():
        tr = NJ * NW + wid
        for hb, tt in _pre:
            pltpu.async_copy(hb.at[pl.ds(tr, 1)], tt.at[pl.ds(NJ, 1)], psem)
    @pl.when(sid == NS - 1)
    def _():
        pltpu.async_copy(zeros_hbm.at[pl.ds(TAIL_BASE, TAIL_ROWS)],
                         acc_sh.at[pl.ds(TAIL_BASE, TAIL_ROWS)], zsem)
    nr = NJ + jnp.where(wid < NREM, 1, 0)
    for hb, tt in _pre:
        pltpu.make_async_copy(hb.at[pl.ds(row0, NJ)],
                              tt.at[pl.ds(0, NJ)], psem).wait()
    @pl.when(wid < NREM)
    def _():
        tr = NJ * NW + wid
        for hb, tt in _pre:
            pltpu.make_async_copy(hb.at[pl.ds(tr, 1)],
                                  tt.at[pl.ds(NJ, 1)], psem).wait()

    bufs = ((rows0, gsem0, ssem0), (rows1, gsem1, ssem1),
            (rows2, gsem2, ssem2), (rows3, gsem3, ssem3))

    def g_issue(j, b):
        pltpu.async_copy(mat_hbm.at[src_t.at[j]], bufs[b][0], bufs[b][1])

    def g_wait(j, b):
        pltpu.make_async_copy(
            mat_hbm.at[src_t.at[j]], bufs[b][0], bufs[b][1]).wait()

    def s_issue(j, b):
        pltpu.async_copy(bufs[b][0], acc_sh.at[dst_t.at[j]], bufs[b][2],
                         add=True)

    def s_wait(j, b):
        pltpu.make_async_copy(
            bufs[b][0], acc_sh.at[dst_t.at[j]], bufs[b][2]).wait()

    def scale(j, b):
        rows = bufs[b][0]
        @plsc.parallel_loop(0, CHUNK // 16, 1, unroll=2)
        def _(g):
            for l in range(16):
                bval = plsc.load_gather(
                    vals_t, [jnp.broadcast_to(j, (16,)),
                             jnp.broadcast_to(g * 16 + l, (16,))])
                e = g * 16 + l
                for s in range(B // 16):
                    rows[e, pl.ds(s * 16, 16)] = (
                        rows[e, pl.ds(s * 16, 16)] * bval)

    g_issue(0, 0)
    g_issue(1, 1)

    pltpu.make_async_copy(
        zeros_hbm.at[pl.ds(sid * ROWS_PER_TILE, ROWS_PER_TILE)],
        acc_sh.at[pl.ds(sid * ROWS_PER_TILE, ROWS_PER_TILE)], zsem).wait()
    @pl.when(sid == NS - 1)
    def _():
        pltpu.make_async_copy(zeros_hbm.at[pl.ds(TAIL_BASE, TAIL_ROWS)],
                              acc_sh.at[pl.ds(TAIL_BASE, TAIL_ROWS)],
                              zsem).wait()
    plsc.subcore_barrier()

    def quad_body(i, carry):
        for b in range(4):
            j = i * 4 + b
            @pl.when(j < nr)
            def _():
                g_wait(j, b)
                scale(j, b)
                s_issue(j, b)
                jf = j + 2
                bf = (b + 2) % 4
                @pl.when(jf < nr)
                def _():
                    @pl.when(jf >= 4)
                    def _():
                        s_wait(jf - 4, bf)
                    g_issue(jf, bf)
        return carry
    lax.fori_loop(0, (NJ + 4) // 4, quad_body, 0)

    for b in range(4):
        s_wait(0, b)

    plsc.subcore_barrier()
    pltpu.sync_copy(acc_sh.at[pl.ds(sid * ROWS_PER_TILE, ROWS_PER_TILE)],
                    out_hbm.at[cid, pl.ds(sid * ROWS_PER_TILE, ROWS_PER_TILE)])
    @pl.when(sid == NS - 1)
    def _():
        pltpu.sync_copy(acc_sh.at[pl.ds(TAIL_BASE, TAIL_ROWS)],
                        out_hbm.at[cid, pl.ds(TAIL_BASE, TAIL_ROWS)])


def _spmm_partials(mat, src, dst, vals, zeros):
    f = functools.partial(
        pl.kernel,
        out_type=jax.ShapeDtypeStruct((NC, V, B), jnp.float32),
        mesh=plsc.VectorSubcoreMesh(core_axis_name="c", subcore_axis_name="s",
                                    num_cores=NC, num_subcores=NS),
        compiler_params=pltpu.CompilerParams(needs_layout_passes=False,
                                             use_tc_tiling_on_sc=False),
        scratch_types=[
            pltpu.VMEM((NJ + 1, CHUNK), jnp.int32),
            pltpu.VMEM((NJ + 1, CHUNK), jnp.int32),
            pltpu.VMEM((NJ + 1, CHUNK), jnp.float32),
            pltpu.VMEM((CHUNK, B), jnp.float32),
            pltpu.VMEM((CHUNK, B), jnp.float32),
            pltpu.VMEM((CHUNK, B), jnp.float32),
            pltpu.VMEM((CHUNK, B), jnp.float32),
            pltpu.SemaphoreType.DMA,
            pltpu.SemaphoreType.DMA,
            pltpu.SemaphoreType.DMA,
            pltpu.SemaphoreType.DMA,
            pltpu.SemaphoreType.DMA,
            pltpu.SemaphoreType.DMA,
            pltpu.SemaphoreType.DMA,
            pltpu.SemaphoreType.DMA,
            pltpu.SemaphoreType.DMA,
            pltpu.SemaphoreType.DMA,
            pltpu.VMEM_SHARED((V, B), jnp.float32),
        ],
    )(_spmm_body)
    return f(mat, src, dst, vals, zeros)


VR = V * B // 128


def _combine1_body(p_ref, x_ref, o_ref):
    o_ref[...] = p_ref[0] + p_ref[1] - x_ref[...]


def _combine2_body(p_ref, x1_ref, x0_ref, o_ref):
    o_ref[...] = 2.0 * (p_ref[0] + p_ref[1]) - 2.0 * x1_ref[...] - x0_ref[...]


def _combine1(p, x0):
    return pl.pallas_call(
        _combine1_body,
        out_shape=jax.ShapeDtypeStruct((VR, 128), jnp.float32),
    )(p.reshape(2, VR, 128), x0)


def _combine2(p, x1, x0):
    return pl.pallas_call(
        _combine2_body,
        out_shape=jax.ShapeDtypeStruct((VR, 128), jnp.float32),
    )(p.reshape(2, VR, 128), x1, x0)


def _dotT(a_pb, w_pn):
    return lax.dot_general(a_pb, w_pn, (((0,), (0,)), ((), ())),
                           preferred_element_type=jnp.float32)


def _epilogue_body(x0, x1, x2, x3, p4, Wc, bc, Wf, bf, gam, bet,
                   Wl1, bl1, Wl2, bl2, o_ref, y_ref):
    xv = [x[...] for x in (x0, x1, x2, x3)]
    xv.append(2.0 * (p4[0] + p4[1]) - 2.0 * xv[3] - xv[2])
    hps = []
    for f in range(K):
        h = jnp.zeros((VR, 128), jnp.float32) + bc[f]
        for k in range(K):
            h = h + Wc[k, f] * xv[k]
        h = jnp.maximum(h, 0.0)
        red = jnp.max(h.reshape(1250, 4, 128), axis=1)
        hp = jnp.maximum(red[:, :B], red[:, B:])
        hps.append(hp.reshape(10, 125, B))
    hpall = jnp.concatenate(hps, axis=1)
    for c in range(10):
        y_ref[c] = _dotT(hpall[c], Wf[...]) + bf[...]
    z = y_ref[...].reshape(10 * B, HID)
    xj = None
    for i in range(NUM_LAYERS):
        mean = jnp.mean(z, axis=0)
        var = jnp.mean((z - mean[None, :]) ** 2, axis=0)
        z = (z - mean[None, :]) * lax.rsqrt(var[None, :] + 1e-5)
        z = z * gam[i][None, :] + bet[i][None, :]
        z = jnp.maximum(z, 0.0)
        xj = z if xj is None else jnp.maximum(xj, z)
    o = jnp.dot(xj, Wl1[...], preferred_element_type=jnp.float32) + bl1[...][None, :]
    o = jnp.maximum(o, 0.0)
    o = jnp.dot(o, Wl2[...], preferred_element_type=jnp.float32) + bl2[...][None, :]
    m = jnp.max(o, axis=1, keepdims=True)
    lse = jnp.log(jnp.sum(jnp.exp(o - m), axis=1, keepdims=True)) + m
    o_ref[...] = o - lse


def _epilogue(xstack, Wc, bc, Wf, bf, gam, bet, Wl1, bl1, Wl2, bl2):
    smem = pl.BlockSpec(memory_space=pltpu.SMEM)
    vmem = pl.BlockSpec(memory_space=pltpu.VMEM)
    return pl.pallas_call(
        _epilogue_body,
        out_shape=jax.ShapeDtypeStruct((10 * B, OUT), jnp.float32),
        in_specs=[vmem] * 5 + [smem, smem] + [vmem] * 8,
        out_specs=vmem,
        scratch_shapes=[pltpu.VMEM((10, B, HID), jnp.float32)],
    )(*xstack, Wc, bc, Wf, bf, gam, bet, Wl1, bl1, Wl2, bl2)


def kernel(x, L_indices, L_values, W_cl1, b_cl1, W_fc1, b_fc1,
           bn_gamma, bn_beta, W_lin1, b_lin1, W_lin2, b_lin2):
    src = L_indices[1].reshape(NCHUNKS, CHUNK)
    dst = L_indices[0].reshape(NCHUNKS, CHUNK)
    L_values = L_values.reshape(NCHUNKS, CHUNK)
    zeros = jnp.zeros((V, B), jnp.float32)

    x0 = jnp.transpose(x, (1, 0))
    x0v = x0.reshape(VR, 128)
    p = _spmm_partials(x0, src, dst, L_values, zeros)
    x1v = _combine1(p, x0v)
    p = _spmm_partials(x1v.reshape(V, B), src, dst, L_values, zeros)
    x2v = _combine2(p, x1v, x0v)
    p = _spmm_partials(x2v.reshape(V, B), src, dst, L_values, zeros)
    x3v = _combine2(p, x2v, x1v)
    p4 = _spmm_partials(x3v.reshape(V, B), src, dst, L_values, zeros)

    Wf = jnp.transpose(W_fc1.reshape(125, K, HID), (1, 0, 2)).reshape(
        K * 125, HID)
    o = _epilogue((x0v, x1v, x2v, x3v, p4.reshape(2, VR, 128)),
                  W_cl1, b_cl1, Wf, b_fc1,
                  bn_gamma, bn_beta, W_lin1, b_lin1, W_lin2, b_lin2)
    return o.reshape(10, B, OUT).transpose(1, 0, 2).reshape(10 * B, OUT)
